# baseline (speedup 1.0000x reference)
import functools

import jax
import jax.numpy as jnp
from jax import lax
from jax.experimental import pallas as pl
from jax.experimental.pallas import tpu as pltpu
from jax.experimental.pallas import tpu_sc as plsc

NC = 2
NS = 16
LANES = 16
CB = 128
SPC = 2
NBUF = 4
PF = 2


def _sc_layer_body(kch, npt, sup_hbm, srcr, dstr, wr, out_hbm,
                   src_v, dst_v, w_v, rows_v, acc_sh, gsem, ssem):
  c = lax.axis_index("c")
  s = lax.axis_index("s")

  pltpu.sync_copy(srcr.at[c, s], src_v)
  pltpu.sync_copy(dstr.at[c, s], dst_v)
  pltpu.sync_copy(wr.at[c, s], w_v)

  zeros = jnp.zeros((LANES,), jnp.float32)

  def zfill(i, carry):
    rows_v[0, 0, i, :] = zeros
    return carry

  lax.fori_loop(0, CB, zfill, 0)
  for k in range(npt // CB):
    pltpu.sync_copy(rows_v.at[0, 0], acc_sh.at[pl.ds(s * npt + k * CB, CB)])

  def gstart(j, p):
    for t in range(SPC):
      pltpu.async_copy(sup_hbm.at[src_v.at[j, t]], rows_v.at[p, t],
                       gsem.at[p])

  def gwait(j, p):
    for t in range(SPC):
      pltpu.make_async_copy(sup_hbm.at[src_v.at[j, t]], rows_v.at[p, t],
                            gsem.at[p]).wait()

  def sstart(j, p):
    for t in range(SPC):
      pltpu.async_copy(rows_v.at[p, t], acc_sh.at[dst_v.at[j, t]],
                       ssem.at[p], add=True)

  def swait(j, p):
    for t in range(SPC):
      pltpu.make_async_copy(rows_v.at[p, t], acc_sh.at[dst_v.at[j, t]],
                            ssem.at[p]).wait()

  for j in range(PF):
    gstart(j, j)
  plsc.subcore_barrier()

  def chunk(j, carry):
    p = lax.rem(j, NBUF)

    @pl.when(j + PF < kch)
    def _():
      p2 = lax.rem(j + PF, NBUF)

      @pl.when(j >= NBUF - PF)
      def _():
        swait(j - (NBUF - PF), p2)

      gstart(j + PF, p2)

    gwait(j, p)
    for t in range(SPC):
      for g in range(CB // LANES):
        wv = w_v[j, t, pl.ds(g * LANES, LANES)]
        base = g * LANES
        for k in range(LANES):
          rows_v[p, t, base + k, :] = rows_v[p, t, base + k, :] * wv[k]
    sstart(j, p)
    return carry

  lax.fori_loop(0, kch, chunk, 0)
  for j in range(max(0, kch - NBUF), kch):
    swait(j, j % NBUF)
  plsc.subcore_barrier()

  pltpu.sync_copy(acc_sh.at[pl.ds(s * npt, npt)],
                  out_hbm.at[c, pl.ds(s * npt, npt)])


def _make_sc_layer(kch, npt, n_pad):
  mesh = plsc.VectorSubcoreMesh(core_axis_name="c", subcore_axis_name="s",
                                num_cores=NC, num_subcores=NS)
  return pl.kernel(
      functools.partial(_sc_layer_body, kch, npt),
      out_type=jax.ShapeDtypeStruct((NC, n_pad, LANES), jnp.float32),
      mesh=mesh,
      scratch_types=[
          pltpu.VMEM((kch, SPC, CB), jnp.int32),
          pltpu.VMEM((kch, SPC, CB), jnp.int32),
          pltpu.VMEM((kch, SPC, CB), jnp.float32),
          pltpu.VMEM((NBUF, SPC, CB, LANES), jnp.float32),
          pltpu.VMEM_SHARED((n_pad, LANES), jnp.float32),
          pltpu.SemaphoreType.DMA((NBUF,)),
          pltpu.SemaphoreType.DMA((NBUF,)),
      ],
      compiler_params=pltpu.CompilerParams(use_tc_tiling_on_sc=False),
  )


def _mm_body(x_ref, w_ref, o_ref):
  o_ref[...] = jnp.dot(x_ref[...], w_ref[...],
                       preferred_element_type=jnp.float32)


def _combine1_body(p_ref, b_ref, w_ref, o_ref):
  h = jnp.maximum(p_ref[0] + p_ref[1] + b_ref[...], 0.0)
  o_ref[...] = jnp.dot(h, w_ref[...], preferred_element_type=jnp.float32)


def _combine2_body(p_ref, b_ref, o_ref):
  o_ref[...] = p_ref[0] + p_ref[1] + b_ref[...]


def kernel(feature, edge_index, edge_weight, W1, b1, W2, b2):
  n, d = feature.shape
  h = W1.shape[1]
  cdim = W2.shape[1]
  e = edge_weight.shape[0]

  npt = ((n + NS * CB - 1) // (NS * CB)) * CB
  n_pad = NS * npt
  slot = SPC * CB
  kch = (e + NC * NS * slot - 1) // (NC * NS * slot)
  e_pad = NC * NS * kch * slot

  src = edge_index[0]
  dst = edge_index[1]
  pad = e_pad - e
  srcr = jnp.pad(src, (0, pad)).reshape(NC, NS, kch, SPC, CB)
  dstr = jnp.pad(dst, (0, pad)).reshape(NC, NS, kch, SPC, CB)
  wr = jnp.pad(edge_weight, (0, pad)).reshape(NC, NS, kch, SPC, CB)

  w2p = jnp.zeros((h, LANES), jnp.float32).at[:, :cdim].set(W2)
  b1r = b1.reshape(1, h)
  b2p = jnp.zeros((1, LANES), jnp.float32).at[0, :cdim].set(b2)

  rb = 1000
  support1 = pl.pallas_call(
      _mm_body,
      grid=(n // rb,),
      in_specs=[pl.BlockSpec((rb, d), lambda i: (i, 0)),
                pl.BlockSpec((d, h), lambda i: (0, 0))],
      out_specs=pl.BlockSpec((rb, h), lambda i: (i, 0)),
      out_shape=jax.ShapeDtypeStruct((n, h), jnp.float32),
  )(feature, W1)

  sc_layer = _make_sc_layer(kch, npt, n_pad)
  p1 = sc_layer(support1, srcr, dstr, wr)

  rb2 = n_pad // 10
  support2 = pl.pallas_call(
      _combine1_body,
      grid=(10,),
      in_specs=[pl.BlockSpec((NC, rb2, h), lambda i: (0, i, 0)),
                pl.BlockSpec((1, h), lambda i: (0, 0)),
                pl.BlockSpec((h, LANES), lambda i: (0, 0))],
      out_specs=pl.BlockSpec((rb2, LANES), lambda i: (i, 0)),
      out_shape=jax.ShapeDtypeStruct((n_pad, LANES), jnp.float32),
  )(p1, b1r, w2p)

  p2 = sc_layer(support2, srcr, dstr, wr)

  out16 = pl.pallas_call(
      _combine2_body,
      grid=(10,),
      in_specs=[pl.BlockSpec((NC, rb2, LANES), lambda i: (0, i, 0)),
                pl.BlockSpec((1, LANES), lambda i: (0, 0))],
      out_specs=pl.BlockSpec((rb2, LANES), lambda i: (i, 0)),
      out_shape=jax.ShapeDtypeStruct((n_pad, LANES), jnp.float32),
  )(p2, b2p)

  return out16[:n, :cdim]


# speedup vs baseline: 1.2999x; 1.2999x over previous
import functools

import jax
import jax.numpy as jnp
from jax import lax
from jax.experimental import pallas as pl
from jax.experimental.pallas import tpu as pltpu
from jax.experimental.pallas import tpu_sc as plsc

NC = 2
NS = 16
LANES = 16
CB = 128
SPC = 1
NBUF = 8
PF = 4


def _sc_layer_body(kch, npt, sup_hbm, srcr, dstr, wr, out_hbm,
                   src_v, dst_v, w_v, rows_v, acc_sh, gsem, ssem):
  c = lax.axis_index("c")
  s = lax.axis_index("s")

  pltpu.sync_copy(srcr.at[c, s], src_v)
  pltpu.sync_copy(dstr.at[c, s], dst_v)
  pltpu.sync_copy(wr.at[c, s], w_v)

  zeros = jnp.zeros((LANES,), jnp.float32)

  def zfill(i, carry):
    rows_v[0, 0, i, :] = zeros
    return carry

  lax.fori_loop(0, CB, zfill, 0)
  for k in range(npt // CB):
    pltpu.sync_copy(rows_v.at[0, 0], acc_sh.at[pl.ds(s * npt + k * CB, CB)])

  def gstart(j, p):
    for t in range(SPC):
      pltpu.async_copy(sup_hbm.at[src_v.at[j, t]], rows_v.at[p, t],
                       gsem.at[p])

  def gwait(j, p):
    for t in range(SPC):
      pltpu.make_async_copy(sup_hbm.at[src_v.at[j, t]], rows_v.at[p, t],
                            gsem.at[p]).wait()

  def sstart(j, p):
    for t in range(SPC):
      pltpu.async_copy(rows_v.at[p, t], acc_sh.at[dst_v.at[j, t]],
                       ssem.at[p], add=True)

  def swait(j, p):
    for t in range(SPC):
      pltpu.make_async_copy(rows_v.at[p, t], acc_sh.at[dst_v.at[j, t]],
                            ssem.at[p]).wait()

  for j in range(PF):
    gstart(j, j)
  plsc.subcore_barrier()

  def chunk(j, carry):
    p = lax.rem(j, NBUF)

    @pl.when(j + PF < kch)
    def _():
      p2 = lax.rem(j + PF, NBUF)

      @pl.when(j >= NBUF - PF)
      def _():
        swait(j - (NBUF - PF), p2)

      gstart(j + PF, p2)

    gwait(j, p)
    for t in range(SPC):
      for g in range(CB // LANES):
        wv = w_v[j, t, pl.ds(g * LANES, LANES)]
        base = g * LANES
        for k in range(LANES):
          rows_v[p, t, base + k, :] = rows_v[p, t, base + k, :] * wv[k]
    sstart(j, p)
    return carry

  lax.fori_loop(0, kch, chunk, 0)
  for j in range(max(0, kch - NBUF), kch):
    swait(j, j % NBUF)
  plsc.subcore_barrier()

  pltpu.sync_copy(acc_sh.at[pl.ds(s * npt, npt)],
                  out_hbm.at[c, pl.ds(s * npt, npt)])


def _make_sc_layer(kch, npt, n_pad):
  mesh = plsc.VectorSubcoreMesh(core_axis_name="c", subcore_axis_name="s",
                                num_cores=NC, num_subcores=NS)
  return pl.kernel(
      functools.partial(_sc_layer_body, kch, npt),
      out_type=jax.ShapeDtypeStruct((NC, n_pad, LANES), jnp.float32),
      mesh=mesh,
      scratch_types=[
          pltpu.VMEM((kch, SPC, CB), jnp.int32),
          pltpu.VMEM((kch, SPC, CB), jnp.int32),
          pltpu.VMEM((kch, SPC, CB), jnp.float32),
          pltpu.VMEM((NBUF, SPC, CB, LANES), jnp.float32),
          pltpu.VMEM_SHARED((n_pad, LANES), jnp.float32),
          pltpu.SemaphoreType.DMA((NBUF,)),
          pltpu.SemaphoreType.DMA((NBUF,)),
      ],
      compiler_params=pltpu.CompilerParams(use_tc_tiling_on_sc=False),
  )


def _mm_body(x_ref, w_ref, o_ref):
  o_ref[...] = jnp.dot(x_ref[...], w_ref[...],
                       preferred_element_type=jnp.float32)


def _combine1_body(p_ref, b_ref, w_ref, o_ref):
  h = jnp.maximum(p_ref[0] + p_ref[1] + b_ref[...], 0.0)
  o_ref[...] = jnp.dot(h, w_ref[...], preferred_element_type=jnp.float32)


def _combine2_body(p_ref, b_ref, o_ref):
  o_ref[...] = p_ref[0] + p_ref[1] + b_ref[...]


def kernel(feature, edge_index, edge_weight, W1, b1, W2, b2):
  n, d = feature.shape
  h = W1.shape[1]
  cdim = W2.shape[1]
  e = edge_weight.shape[0]

  npt = ((n + NS * CB - 1) // (NS * CB)) * CB
  n_pad = NS * npt
  slot = SPC * CB
  kch = (e + NC * NS * slot - 1) // (NC * NS * slot)
  e_pad = NC * NS * kch * slot

  src = edge_index[0]
  dst = edge_index[1]
  pad = e_pad - e
  srcr = jnp.pad(src, (0, pad)).reshape(NC, NS, kch, SPC, CB)
  dstr = jnp.pad(dst, (0, pad)).reshape(NC, NS, kch, SPC, CB)
  wr = jnp.pad(edge_weight, (0, pad)).reshape(NC, NS, kch, SPC, CB)

  w2p = jnp.zeros((h, LANES), jnp.float32).at[:, :cdim].set(W2)
  b1r = b1.reshape(1, h)
  b2p = jnp.zeros((1, LANES), jnp.float32).at[0, :cdim].set(b2)

  rb = 1000
  support1 = pl.pallas_call(
      _mm_body,
      grid=(n // rb,),
      in_specs=[pl.BlockSpec((rb, d), lambda i: (i, 0)),
                pl.BlockSpec((d, h), lambda i: (0, 0))],
      out_specs=pl.BlockSpec((rb, h), lambda i: (i, 0)),
      out_shape=jax.ShapeDtypeStruct((n, h), jnp.float32),
  )(feature, W1)

  sc_layer = _make_sc_layer(kch, npt, n_pad)
  p1 = sc_layer(support1, srcr, dstr, wr)

  rb2 = n_pad // 10
  support2 = pl.pallas_call(
      _combine1_body,
      grid=(10,),
      in_specs=[pl.BlockSpec((NC, rb2, h), lambda i: (0, i, 0)),
                pl.BlockSpec((1, h), lambda i: (0, 0)),
                pl.BlockSpec((h, LANES), lambda i: (0, 0))],
      out_specs=pl.BlockSpec((rb2, LANES), lambda i: (i, 0)),
      out_shape=jax.ShapeDtypeStruct((n_pad, LANES), jnp.float32),
  )(p1, b1r, w2p)

  p2 = sc_layer(support2, srcr, dstr, wr)

  out16 = pl.pallas_call(
      _combine2_body,
      grid=(10,),
      in_specs=[pl.BlockSpec((NC, rb2, LANES), lambda i: (0, i, 0)),
                pl.BlockSpec((1, LANES), lambda i: (0, 0))],
      out_specs=pl.BlockSpec((rb2, LANES), lambda i: (i, 0)),
      out_shape=jax.ShapeDtypeStruct((n_pad, LANES), jnp.float32),
  )(p2, b2p)

  return out16[:n, :cdim]
